# baseline (speedup 1.0000x reference)
import math

import jax
import jax.numpy as jnp
from jax import lax
from jax.experimental import pallas as pl
from jax.experimental.pallas import tpu as pltpu
from jax.experimental.pallas import tpu_sc as plsc

N = 10000
E = 160000
HID = 256
EDGE = 128
H = 8
D = HID // H

NC = 2
NS = 16
NW = NC * NS

F32 = jnp.float32
BF16 = jnp.bfloat16


def _ln(x, g, b):
    m = jnp.mean(x, axis=-1, keepdims=True)
    xm = x - m
    v = jnp.mean(xm * xm, axis=-1, keepdims=True)
    return xm * lax.rsqrt(v + 1e-5) * g + b


BN = 400


def _node_body(x_ref, g_ref, b_ref, wq_ref, wk_ref, wv_ref, q_ref, k_ref, v_ref):
    h = _ln(x_ref[...], g_ref[...], b_ref[...])
    q_ref[...] = jnp.dot(h, wq_ref[...], preferred_element_type=F32).astype(BF16)
    k_ref[...] = jnp.dot(h, wk_ref[...], preferred_element_type=F32).astype(BF16)
    v_ref[...] = jnp.dot(h, wv_ref[...], preferred_element_type=F32).astype(BF16)


def _tc_nodes(x, n1_g, n1_b, WqT, WkT, WvT):
    row = pl.BlockSpec((BN, HID), lambda i: (i, 0))
    full = pl.BlockSpec((HID, HID), lambda i: (0, 0))
    vec = pl.BlockSpec((1, HID), lambda i: (0, 0))
    return pl.pallas_call(
        _node_body,
        grid=(N // BN,),
        in_specs=[row, vec, vec, full, full, full],
        out_specs=[row, row, row],
        out_shape=[jax.ShapeDtypeStruct((N, HID), BF16)] * 3,
    )(x, n1_g.reshape(1, HID), n1_b.reshape(1, HID), WqT, WkT, WvT)


EPW = E // NW
GB = 200
GCHUNKS = EPW // GB


def _sc_gather_body(q_hbm, k_hbm, v_hbm, dst_hbm, src_hbm,
                    qg_hbm, kg_hbm, vg_hbm, idx_v, rows_v, sem):
    c = lax.axis_index("c")
    s = lax.axis_index("s")
    base0 = (s * NC + c) * EPW

    def chunk(i, carry):
        base = base0 + i * GB
        pltpu.sync_copy(dst_hbm.at[pl.ds(base, GB)], idx_v)
        pltpu.async_copy(q_hbm.at[idx_v], rows_v, sem).wait()
        pltpu.sync_copy(rows_v, qg_hbm.at[pl.ds(base, GB)])
        pltpu.sync_copy(src_hbm.at[pl.ds(base, GB)], idx_v)
        pltpu.async_copy(k_hbm.at[idx_v], rows_v, sem).wait()
        pltpu.sync_copy(rows_v, kg_hbm.at[pl.ds(base, GB)])
        pltpu.async_copy(v_hbm.at[idx_v], rows_v, sem).wait()
        pltpu.sync_copy(rows_v, vg_hbm.at[pl.ds(base, GB)])
        return carry

    lax.fori_loop(0, GCHUNKS, chunk, 0)


def _as_i32(t):
    return lax.bitcast_convert_type(t.reshape(t.shape[0], HID // 2, 2), jnp.int32)


def _as_bf16(t):
    return lax.bitcast_convert_type(t, BF16).reshape(t.shape[0], HID)


def _sc_gather(Q, K, V, dst, src):
    f = pl.kernel(
        _sc_gather_body,
        out_type=[jax.ShapeDtypeStruct((E, HID // 2), jnp.int32)] * 3,
        mesh=plsc.VectorSubcoreMesh(core_axis_name="c", subcore_axis_name="s"),
        scratch_types=[
            pltpu.VMEM((GB,), jnp.int32),
            pltpu.VMEM((GB, HID // 2), jnp.int32),
            pltpu.SemaphoreType.DMA,
        ],
    )
    qg, kg, vg = f(_as_i32(Q), _as_i32(K), _as_i32(V), dst, src)
    return _as_bf16(qg), _as_bf16(kg), _as_bf16(vg)


BE = 1600


def _edge_body(ea_ref, qg_ref, kg_ref, vg_ref, w1_ref, b1_ref, w2_ref, b2_ref,
               msg_ref):
    t = jnp.dot(ea_ref[...], w1_ref[...], preferred_element_type=F32) + b1_ref[...]
    t = t * jax.nn.sigmoid(t)
    ew = jnp.dot(t, w2_ref[...], preferred_element_type=F32) + b2_ref[...]
    s = qg_ref[...].astype(F32) * kg_ref[...].astype(F32) * ew
    lane = lax.broadcasted_iota(jnp.int32, (HID, H), 0)
    head = lax.broadcasted_iota(jnp.int32, (HID, H), 1)
    G = (lane // D == head).astype(F32)
    hs = jnp.dot(s, G, preferred_element_type=F32) * (1.0 / math.sqrt(D))
    attn = jax.nn.sigmoid(hs)
    alane = jnp.dot(attn, G.T, preferred_element_type=F32)
    msg_ref[...] = alane * vg_ref[...].astype(F32) * ew


def _tc_edges(edge_attr, qg, kg, vg, w1T, b1, w2T, b2):
    erow = pl.BlockSpec((BE, EDGE), lambda i: (i, 0))
    hrow = pl.BlockSpec((BE, HID), lambda i: (i, 0))
    w1s = pl.BlockSpec((EDGE, HID), lambda i: (0, 0))
    w2s = pl.BlockSpec((HID, HID), lambda i: (0, 0))
    vec = pl.BlockSpec((1, HID), lambda i: (0, 0))
    return pl.pallas_call(
        _edge_body,
        grid=(E // BE,),
        in_specs=[erow, hrow, hrow, hrow, w1s, vec, w2s, vec],
        out_specs=hrow,
        out_shape=jax.ShapeDtypeStruct((E, HID), F32),
    )(edge_attr, qg, kg, vg, w1T, b1.reshape(1, HID), w2T, b2.reshape(1, HID))


EPS = E // NS
SB = 200
SCHUNKS = EPS // SB
NPAD = 10240
NPS = NPAD // NS
CW = HID // NC


def _sc_scatter_body(msg_hbm, dst_hbm, zeros_hbm, agg_hbm, idx_v, buf_v, shared):
    c = lax.axis_index("c")
    s = lax.axis_index("s")
    pltpu.sync_copy(zeros_hbm.at[pl.ds(s * NPS, NPS)], shared.at[pl.ds(s * NPS, NPS)])
    plsc.subcore_barrier()
    base0 = s * EPS
    col = c * CW

    def chunk(i, carry):
        base = base0 + i * SB
        pltpu.sync_copy(dst_hbm.at[pl.ds(base, SB)], idx_v)
        pltpu.sync_copy(msg_hbm.at[pl.ds(base, SB), pl.ds(col, CW)], buf_v)
        pltpu.sync_copy(buf_v, shared.at[idx_v], add=True)
        return carry

    lax.fori_loop(0, SCHUNKS, chunk, 0)
    plsc.subcore_barrier()
    pltpu.sync_copy(shared.at[pl.ds(s * NPS, NPS)],
                    agg_hbm.at[pl.ds(s * NPS, NPS), pl.ds(col, CW)])


def _sc_scatter(msg, dst):
    zeros = jnp.zeros((NPAD, CW), F32)
    f = pl.kernel(
        _sc_scatter_body,
        out_type=jax.ShapeDtypeStruct((NPAD, HID), F32),
        mesh=plsc.VectorSubcoreMesh(core_axis_name="c", subcore_axis_name="s"),
        scratch_types=[
            pltpu.VMEM((SB,), jnp.int32),
            pltpu.VMEM((SB, CW), F32),
            pltpu.VMEM_SHARED((NPAD, CW), F32),
        ],
    )
    return f(msg, dst, zeros)[:N]


def _final_body(x_ref, agg_ref, ow_ref, ob_ref, g2_ref, b2_ref,
                fw1_ref, fb1_ref, fw2_ref, fb2_ref, out_ref):
    y = x_ref[...] + jnp.dot(agg_ref[...], ow_ref[...], preferred_element_type=F32) + ob_ref[...]
    h2 = _ln(y, g2_ref[...], b2_ref[...])
    ff = jnp.dot(h2, fw1_ref[...], preferred_element_type=F32) + fb1_ref[...]
    ff = 0.5 * ff * (1.0 + lax.erf(ff * (1.0 / math.sqrt(2.0))))
    ff = jnp.dot(ff, fw2_ref[...], preferred_element_type=F32) + fb2_ref[...]
    out_ref[...] = y + ff


def _tc_final(x, agg, out_wT, out_b, n2_g, n2_b, ff_w1T, ff_b1, ff_w2T, ff_b2):
    row = pl.BlockSpec((BN, HID), lambda i: (i, 0))
    full = pl.BlockSpec((HID, HID), lambda i: (0, 0))
    vec = pl.BlockSpec((1, HID), lambda i: (0, 0))
    w1s = pl.BlockSpec((HID, 4 * HID), lambda i: (0, 0))
    v1s = pl.BlockSpec((1, 4 * HID), lambda i: (0, 0))
    w2s = pl.BlockSpec((4 * HID, HID), lambda i: (0, 0))
    return pl.pallas_call(
        _final_body,
        grid=(N // BN,),
        in_specs=[row, row, full, vec, vec, vec, w1s, v1s, w2s, vec],
        out_specs=row,
        out_shape=jax.ShapeDtypeStruct((N, HID), F32),
    )(x, agg, out_wT, out_b.reshape(1, HID), n2_g.reshape(1, HID),
      n2_b.reshape(1, HID), ff_w1T, ff_b1.reshape(1, 4 * HID), ff_w2T,
      ff_b2.reshape(1, HID))


def kernel(x, edge_index, edge_attr, Wq, Wk, Wv, ep_w1, ep_b1, ep_w2, ep_b2,
           out_w, out_b, n1_g, n1_b, n2_g, n2_b, ff_w1, ff_b1, ff_w2, ff_b2):
    src = edge_index[0]
    dst = edge_index[1]
    Q, K, V = _tc_nodes(x, n1_g, n1_b, Wq.T, Wk.T, Wv.T)
    qg, kg, vg = _sc_gather(Q, K, V, dst, src)
    msg = _tc_edges(edge_attr, qg, kg, vg, ep_w1.T, ep_b1, ep_w2.T, ep_b2)
    agg = _sc_scatter(msg, dst)
    return _tc_final(x, agg, out_w.T, out_b, n2_g, n2_b,
                     ff_w1.T, ff_b1, ff_w2.T, ff_b2)


# speedup vs baseline: 4.0656x; 4.0656x over previous
import math

import jax
import jax.numpy as jnp
from jax import lax
from jax.experimental import pallas as pl
from jax.experimental.pallas import tpu as pltpu
from jax.experimental.pallas import tpu_sc as plsc

N = 10000
E = 160000
HID = 256
EDGE = 128
H = 8
D = HID // H

NC = 2
NS = 16
NW = NC * NS

F32 = jnp.float32
BF16 = jnp.bfloat16


def _ln(x, g, b):
    m = jnp.mean(x, axis=-1, keepdims=True)
    xm = x - m
    v = jnp.mean(xm * xm, axis=-1, keepdims=True)
    return xm * lax.rsqrt(v + 1e-5) * g + b


BN = 400
HH = HID // 2

_M16 = -65536


def _pack_bf16_pairs(t):
    e = lax.bitcast_convert_type(t[:, :HH], jnp.int32)
    o = lax.bitcast_convert_type(t[:, HH:], jnp.int32)
    e16 = lax.shift_right_logical(e + 0x8000, 16)
    o16 = (o + 0x8000) & _M16
    return o16 | e16


def _unpack_bf16_pairs(p):
    e = lax.bitcast_convert_type(lax.shift_left(p, 16), F32)
    o = lax.bitcast_convert_type(p & _M16, F32)
    return e, o


def _node_body(x_ref, g_ref, b_ref, wq_ref, wk_ref, wv_ref, q_ref, k_ref, v_ref):
    h = _ln(x_ref[...], g_ref[...], b_ref[...])
    q_ref[...] = _pack_bf16_pairs(jnp.dot(h, wq_ref[...], preferred_element_type=F32))
    k_ref[...] = _pack_bf16_pairs(jnp.dot(h, wk_ref[...], preferred_element_type=F32))
    v_ref[...] = _pack_bf16_pairs(jnp.dot(h, wv_ref[...], preferred_element_type=F32))


def _tc_nodes(x, n1_g, n1_b, WqT, WkT, WvT):
    row = pl.BlockSpec((BN, HID), lambda i: (i, 0))
    hrow = pl.BlockSpec((BN, HH), lambda i: (i, 0))
    full = pl.BlockSpec((HID, HID), lambda i: (0, 0))
    vec = pl.BlockSpec((1, HID), lambda i: (0, 0))
    return pl.pallas_call(
        _node_body,
        grid=(N // BN,),
        in_specs=[row, vec, vec, full, full, full],
        out_specs=[hrow, hrow, hrow],
        out_shape=[jax.ShapeDtypeStruct((N, HH), jnp.int32)] * 3,
    )(x, n1_g.reshape(1, HID), n1_b.reshape(1, HID), WqT, WkT, WvT)


EPW = E // NW
GB = 200
GCHUNKS = EPW // GB


def _sc_gather_body(q_hbm, k_hbm, v_hbm, dst_hbm, src_hbm,
                    qg_hbm, kg_hbm, vg_hbm, idx_v, rows_v, sem):
    c = lax.axis_index("c")
    s = lax.axis_index("s")
    base0 = (s * NC + c) * EPW

    def chunk(i, carry):
        base = base0 + i * GB
        pltpu.sync_copy(dst_hbm.at[pl.ds(base, GB)], idx_v)
        pltpu.async_copy(q_hbm.at[idx_v], rows_v, sem).wait()
        pltpu.sync_copy(rows_v, qg_hbm.at[pl.ds(base, GB)])
        pltpu.sync_copy(src_hbm.at[pl.ds(base, GB)], idx_v)
        pltpu.async_copy(k_hbm.at[idx_v], rows_v, sem).wait()
        pltpu.sync_copy(rows_v, kg_hbm.at[pl.ds(base, GB)])
        pltpu.async_copy(v_hbm.at[idx_v], rows_v, sem).wait()
        pltpu.sync_copy(rows_v, vg_hbm.at[pl.ds(base, GB)])
        return carry

    lax.fori_loop(0, GCHUNKS, chunk, 0)


def _sc_gather(Q, K, V, dst, src):
    f = pl.kernel(
        _sc_gather_body,
        out_type=[jax.ShapeDtypeStruct((E, HH), jnp.int32)] * 3,
        mesh=plsc.VectorSubcoreMesh(core_axis_name="c", subcore_axis_name="s"),
        scratch_types=[
            pltpu.VMEM((GB,), jnp.int32),
            pltpu.VMEM((GB, HH), jnp.int32),
            pltpu.SemaphoreType.DMA,
        ],
    )
    return f(Q, K, V, dst, src)


BE = 1600


def _edge_body(ea_ref, qg_ref, kg_ref, vg_ref, w1_ref, b1_ref, w2_ref, b2_ref,
               msg_ref):
    t = jnp.dot(ea_ref[...], w1_ref[...], preferred_element_type=F32) + b1_ref[...]
    t = t * jax.nn.sigmoid(t)
    ew = jnp.dot(t, w2_ref[...], preferred_element_type=F32) + b2_ref[...]
    ew_e, ew_o = ew[:, :HH], ew[:, HH:]
    q_e, q_o = _unpack_bf16_pairs(qg_ref[...])
    k_e, k_o = _unpack_bf16_pairs(kg_ref[...])
    v_e, v_o = _unpack_bf16_pairs(vg_ref[...])
    s = q_e * k_e * ew_e + q_o * k_o * ew_o
    lane = lax.broadcasted_iota(jnp.int32, (HH, H), 0)
    head = lax.broadcasted_iota(jnp.int32, (HH, H), 1)
    G = (lane // (D // 2) == head).astype(F32)
    hs = jnp.dot(s, G, preferred_element_type=F32) * (1.0 / math.sqrt(D))
    attn = jax.nn.sigmoid(hs)
    alane = jnp.dot(attn, G.T, preferred_element_type=F32)
    msg_ref[:, :HH] = alane * v_e * ew_e
    msg_ref[:, HH:] = alane * v_o * ew_o


def _tc_edges(edge_attr, qg, kg, vg, w1T, b1, w2Tp, b2p):
    erow = pl.BlockSpec((BE, EDGE), lambda i: (i, 0))
    irow = pl.BlockSpec((BE, HH), lambda i: (i, 0))
    hrow = pl.BlockSpec((BE, HID), lambda i: (i, 0))
    w1s = pl.BlockSpec((EDGE, HID), lambda i: (0, 0))
    w2s = pl.BlockSpec((HID, HID), lambda i: (0, 0))
    vec = pl.BlockSpec((1, HID), lambda i: (0, 0))
    return pl.pallas_call(
        _edge_body,
        grid=(E // BE,),
        in_specs=[erow, irow, irow, irow, w1s, vec, w2s, vec],
        out_specs=hrow,
        out_shape=jax.ShapeDtypeStruct((E, HID), F32),
    )(edge_attr, qg, kg, vg, w1T, b1.reshape(1, HID), w2Tp, b2p.reshape(1, HID))


EPS = E // NS
SB = 200
SCHUNKS = EPS // SB
NPAD = 10240
NPS = NPAD // NS
CW = HID // NC


def _sc_scatter_body(msg_hbm, dst_hbm, zeros_hbm, agg_hbm, idx_v, buf_v, shared):
    c = lax.axis_index("c")
    s = lax.axis_index("s")
    pltpu.sync_copy(zeros_hbm.at[pl.ds(s * NPS, NPS)], shared.at[pl.ds(s * NPS, NPS)])
    plsc.subcore_barrier()
    base0 = s * EPS
    col = c * CW

    def chunk(i, carry):
        base = base0 + i * SB
        pltpu.sync_copy(dst_hbm.at[pl.ds(base, SB)], idx_v)
        pltpu.sync_copy(msg_hbm.at[pl.ds(base, SB), pl.ds(col, CW)], buf_v)
        pltpu.sync_copy(buf_v, shared.at[idx_v], add=True)
        return carry

    lax.fori_loop(0, SCHUNKS, chunk, 0)
    plsc.subcore_barrier()
    pltpu.sync_copy(shared.at[pl.ds(s * NPS, NPS)],
                    agg_hbm.at[pl.ds(s * NPS, NPS), pl.ds(col, CW)])


def _sc_scatter(msg, dst):
    zeros = jnp.zeros((NPAD, CW), F32)
    f = pl.kernel(
        _sc_scatter_body,
        out_type=jax.ShapeDtypeStruct((NPAD, HID), F32),
        mesh=plsc.VectorSubcoreMesh(core_axis_name="c", subcore_axis_name="s"),
        scratch_types=[
            pltpu.VMEM((SB,), jnp.int32),
            pltpu.VMEM((SB, CW), F32),
            pltpu.VMEM_SHARED((NPAD, CW), F32),
        ],
    )
    return f(msg, dst, zeros)[:N]


def _final_body(x_ref, agg_ref, ow_ref, ob_ref, g2_ref, b2_ref,
                fw1_ref, fb1_ref, fw2_ref, fb2_ref, out_ref):
    y = x_ref[...] + jnp.dot(agg_ref[...], ow_ref[...], preferred_element_type=F32) + ob_ref[...]
    h2 = _ln(y, g2_ref[...], b2_ref[...])
    ff = jnp.dot(h2, fw1_ref[...], preferred_element_type=F32) + fb1_ref[...]
    ff = 0.5 * ff * (1.0 + lax.erf(ff * (1.0 / math.sqrt(2.0))))
    ff = jnp.dot(ff, fw2_ref[...], preferred_element_type=F32) + fb2_ref[...]
    out_ref[...] = y + ff


def _tc_final(x, agg, out_wT, out_b, n2_g, n2_b, ff_w1T, ff_b1, ff_w2T, ff_b2):
    row = pl.BlockSpec((BN, HID), lambda i: (i, 0))
    full = pl.BlockSpec((HID, HID), lambda i: (0, 0))
    vec = pl.BlockSpec((1, HID), lambda i: (0, 0))
    w1s = pl.BlockSpec((HID, 4 * HID), lambda i: (0, 0))
    v1s = pl.BlockSpec((1, 4 * HID), lambda i: (0, 0))
    w2s = pl.BlockSpec((4 * HID, HID), lambda i: (0, 0))
    return pl.pallas_call(
        _final_body,
        grid=(N // BN,),
        in_specs=[row, row, full, vec, vec, vec, w1s, v1s, w2s, vec],
        out_specs=row,
        out_shape=jax.ShapeDtypeStruct((N, HID), F32),
    )(x, agg, out_wT, out_b.reshape(1, HID), n2_g.reshape(1, HID),
      n2_b.reshape(1, HID), ff_w1T, ff_b1.reshape(1, 4 * HID), ff_w2T,
      ff_b2.reshape(1, HID))


def kernel(x, edge_index, edge_attr, Wq, Wk, Wv, ep_w1, ep_b1, ep_w2, ep_b2,
           out_w, out_b, n1_g, n1_b, n2_g, n2_b, ff_w1, ff_b1, ff_w2, ff_b2):
    src = edge_index[0]
    dst = edge_index[1]
    perm = jnp.concatenate([jnp.arange(0, HID, 2), jnp.arange(1, HID, 2)])
    Q, K, V = _tc_nodes(x, n1_g, n1_b, Wq.T[:, perm], Wk.T[:, perm], Wv.T[:, perm])
    qg, kg, vg = _sc_gather(Q, K, V, dst, src)
    msg = _tc_edges(edge_attr, qg, kg, vg, ep_w1.T, ep_b1,
                    ep_w2.T[:, perm], ep_b2[perm])
    agg = _sc_scatter(msg, dst)
    return _tc_final(x, agg, out_w.T[perm, :], out_b, n2_g, n2_b,
                     ff_w1.T, ff_b1, ff_w2.T, ff_b2)
